# Initial kernel scaffold; baseline (speedup 1.0000x reference)
import jax
import jax.numpy as jnp
from jax.experimental import pallas as pl


def kernel(x, edge_index, edge_type, rel_embed, w_in, w_out, w_loop, w_rel, loop_rel):
    raise NotImplementedError("write your pallas kernel here")


# speedup vs baseline: 5.2748x; 5.2748x over previous
import functools

import jax
import jax.numpy as jnp
from jax import lax
from jax.experimental import pallas as pl
from jax.experimental.pallas import tpu as pltpu
from jax.experimental.pallas import tpu_sc as plsc

_PI = 3.141592653589793
_NC = 2
_NS = 16
_B = 64
_ZR = 8


def _ceil_to(v, m):
    return (v + m - 1) // m * m


def _sc_degree(grow, Ep, Npad):
    nsl = Npad // _NS
    eps = Ep // _NS
    nblk = eps // _B
    mesh = plsc.VectorSubcoreMesh(core_axis_name="c", subcore_axis_name="s")

    @functools.partial(
        pl.kernel,
        out_type=jax.ShapeDtypeStruct((2 * Npad, 128), jnp.float32),
        mesh=mesh,
        scratch_types=[
            pltpu.VMEM_SHARED((Npad, 128), jnp.float32),
            pltpu.VMEM((_B,), jnp.int32),
            pltpu.VMEM((_B, 128), jnp.float32),
            pltpu.VMEM((_ZR, 128), jnp.float32),
        ],
    )
    def k(grow_hbm, out_hbm, acc, rowv, oneh, zb):
        d = lax.axis_index("c")
        s = lax.axis_index("s")
        lane = lax.iota(jnp.int32, 16)
        one16 = jnp.where(lane == 0, 1.0, 0.0).astype(jnp.float32)
        zero16 = jnp.zeros((16,), jnp.float32)

        @pl.loop(0, _B)
        def _(r):
            oneh[r, pl.ds(0, 16)] = one16
            for k8 in range(1, 8):
                oneh[r, pl.ds(k8 * 16, 16)] = zero16

        @pl.loop(0, _ZR)
        def _(r):
            for k8 in range(8):
                zb[r, pl.ds(k8 * 16, 16)] = zero16

        @pl.loop(0, nsl // _ZR)
        def _(j):
            pltpu.sync_copy(zb, acc.at[pl.ds(s * nsl + j * _ZR, _ZR)])

        plsc.subcore_barrier()

        base = d * Ep + s * eps

        @pl.loop(0, nblk)
        def _(b):
            pltpu.sync_copy(grow_hbm.at[pl.ds(base + b * _B, _B)], rowv)
            pltpu.sync_copy(oneh, acc.at[rowv], add=True)

        plsc.subcore_barrier()
        pltpu.sync_copy(
            acc.at[pl.ds(s * nsl, nsl)],
            out_hbm.at[pl.ds(d * Npad + s * nsl, nsl)],
        )

    return k(grow)


def _tc_prep(degc, x, rel_embed, loop_rel, w_rel, w_loop, Npad, Np2):
    N, D = x.shape
    Dh = D // 2
    R = rel_embed.shape[0]

    def body(degc_ref, x_ref, rel_ref, lrel_ref, wrel_ref, wloop_ref,
             cs_ref, dinv_ref, loopc_ref, relout_ref):
        deg = degc_ref[...][:, 0].reshape(2, Npad)[:, :N]
        dinv = jnp.where(deg > 0, lax.rsqrt(deg), 0.0)
        dinv_ref[...] = jnp.concatenate(
            [dinv, jnp.zeros((2, Np2 - N), jnp.float32)], axis=1)

        relf = jnp.concatenate([rel_ref[...], lrel_ref[...]], axis=0)
        r = relf * (_PI / 1.5)
        cs = jnp.concatenate([jnp.cos(r), jnp.sin(r)], axis=1)
        cs_ref[...] = jnp.concatenate(
            [cs, jnp.zeros((cs_ref.shape[0] - (R + 1), D), jnp.float32)], axis=0)

        xv = x_ref[...]
        cl = jnp.concatenate([cs[R, :Dh], cs[R, :Dh]], axis=0)
        sl = jnp.concatenate([cs[R, Dh:], cs[R, Dh:]], axis=0)
        wl = wloop_ref[...]
        wc = jnp.concatenate([wl[:Dh], -wl[Dh:]], axis=0)
        ws = jnp.concatenate([wl[Dh:], wl[:Dh]], axis=0)
        loopc_ref[...] = (
            jnp.dot(xv * cl[None, :], wc, preferred_element_type=jnp.float32)
            + jnp.dot(xv * sl[None, :], ws, preferred_element_type=jnp.float32))

        relout_ref[...] = jnp.dot(
            relf, wrel_ref[...], preferred_element_type=jnp.float32)[:R]

    Rpad = _ceil_to(R + 1, 8)
    return pl.pallas_call(
        body,
        out_shape=[
            jax.ShapeDtypeStruct((Rpad, D), jnp.float32),
            jax.ShapeDtypeStruct((2, Np2), jnp.float32),
            jax.ShapeDtypeStruct((N, D), jnp.float32),
            jax.ShapeDtypeStruct((R, Dh), jnp.float32),
        ],
    )(degc, x, rel_embed, loop_rel, w_rel, w_loop)


def _tc_xs(x, dinv, Npad, Np2, bn):
    N, D = x.shape

    def body(x_ref, dinv_ref, xs_ref):
        xv = x_ref[...]
        dv = dinv_ref[...]
        for d in range(2):
            xs_ref[d] = xv * dv[d][:, None]

    return pl.pallas_call(
        body,
        grid=(Np2 // bn,),
        in_specs=[
            pl.BlockSpec((bn, D), lambda i: (i, 0)),
            pl.BlockSpec((2, bn), lambda i: (0, i)),
        ],
        out_specs=pl.BlockSpec((2, bn, D), lambda i: (0, i, 0)),
        out_shape=jax.ShapeDtypeStruct((2, Npad, D), jnp.float32),
    )(x, dinv)


def _sc_edges(gcol, grow, gt, xs, cs, Ep, Npad, Dh, Rpad):
    nsl = Npad // _NS
    eps = Ep // _NS
    nblk = eps // _B
    D = 2 * Dh
    mesh = plsc.VectorSubcoreMesh(core_axis_name="c", subcore_axis_name="s")

    @functools.partial(
        pl.kernel,
        out_type=jax.ShapeDtypeStruct((4 * Npad, D), jnp.float32),
        mesh=mesh,
        scratch_types=[
            pltpu.VMEM_SHARED((Npad, D), jnp.float32),
            pltpu.VMEM_SHARED((Rpad, D), jnp.float32),
            pltpu.VMEM((_B,), jnp.int32),
            pltpu.VMEM((_B,), jnp.int32),
            pltpu.VMEM((_B,), jnp.int32),
            pltpu.VMEM((_B, D), jnp.float32),
            pltpu.VMEM((_B, D), jnp.float32),
            pltpu.VMEM((_B, D), jnp.float32),
            pltpu.VMEM((_ZR, D), jnp.float32),
            pltpu.SemaphoreType.DMA,
        ],
    )
    def k(gcol_hbm, grow_hbm, gt_hbm, xs_hbm, cs_hbm, out_hbm,
          acc, css, colv, rowv, tv, xsv, csg, pcs, zb, sem):
        h = lax.axis_index("c")
        s = lax.axis_index("s")

        @pl.when(s == 0)
        def _():
            pltpu.sync_copy(cs_hbm, css)

        zero16 = jnp.zeros((16,), jnp.float32)

        @pl.loop(0, _ZR)
        def _(r):
            for k8 in range(D // 16):
                zb[r, pl.ds(k8 * 16, 16)] = zero16

        def zero_own():
            @pl.loop(0, nsl // _ZR)
            def _(j):
                pltpu.sync_copy(zb, acc.at[pl.ds(s * nsl + j * _ZR, _ZR)])

        zero_own()
        plsc.subcore_barrier()

        def compute(hoff):
            @pl.loop(0, _B)
            def _(e):
                for kk in range(0, Dh, 16):
                    xv = xsv[e, pl.ds(hoff + kk, 16)]
                    cv = csg[e, pl.ds(kk, 16)]
                    sv = csg[e, pl.ds(Dh + kk, 16)]
                    pcs[e, pl.ds(kk, 16)] = xv * cv
                    pcs[e, pl.ds(Dh + kk, 16)] = xv * sv

        for d in range(2):
            @pl.loop(0, nblk)
            def _(b):
                off = d * Ep + s * eps + b * _B
                pltpu.sync_copy(gcol_hbm.at[pl.ds(off, _B)], colv)
                pltpu.sync_copy(grow_hbm.at[pl.ds(off, _B)], rowv)
                pltpu.sync_copy(gt_hbm.at[pl.ds(off, _B)], tv)
                pltpu.async_copy(xs_hbm.at[colv], xsv, sem).wait()
                pltpu.async_copy(css.at[tv], csg, sem).wait()

                @pl.when(h == 0)
                def _():
                    compute(0)

                @pl.when(h == 1)
                def _():
                    compute(Dh)

                pltpu.async_copy(pcs, acc.at[rowv], sem, add=True).wait()

            plsc.subcore_barrier()
            rowoff = (d * 2 + h) * Npad + s * nsl
            pltpu.sync_copy(acc.at[pl.ds(s * nsl, nsl)],
                            out_hbm.at[pl.ds(rowoff, nsl)])

            if d == 0:
                zero_own()
                plsc.subcore_barrier()

    return k(gcol, grow, gt, xs, cs)


def _tc_final(accs, dinv, loopc, w_in, w_out, Npad, Np2, bn):
    N, D = loopc.shape
    Dh = D // 2

    def body(acc_ref, dinv_ref, loopc_ref, win_ref, wout_ref, out_ref):
        a = acc_ref[...]
        dinv = dinv_ref[...]
        res = loopc_ref[...]
        for d, wref in ((0, win_ref), (1, wout_ref)):
            ac = jnp.concatenate([a[d * 2 + 0, :, :Dh], a[d * 2 + 1, :, :Dh]],
                                 axis=1)
            as_ = jnp.concatenate([a[d * 2 + 0, :, Dh:], a[d * 2 + 1, :, Dh:]],
                                  axis=1)
            w = wref[...]
            wc = jnp.concatenate([w[:Dh], -w[Dh:]], axis=0)
            ws = jnp.concatenate([w[Dh:], w[:Dh]], axis=0)
            contrib = (jnp.dot(ac, wc, preferred_element_type=jnp.float32)
                       + jnp.dot(as_, ws, preferred_element_type=jnp.float32))
            res = res + dinv[d][:, None] * contrib
        out_ref[...] = res * (1.0 / 3.0)

    return pl.pallas_call(
        body,
        grid=(Np2 // bn,),
        in_specs=[
            pl.BlockSpec((4, bn, D), lambda i: (0, i, 0)),
            pl.BlockSpec((2, bn), lambda i: (0, i)),
            pl.BlockSpec((bn, D), lambda i: (i, 0)),
            pl.BlockSpec((D, D), lambda i: (0, 0)),
            pl.BlockSpec((D, D), lambda i: (0, 0)),
        ],
        out_specs=pl.BlockSpec((bn, D), lambda i: (i, 0)),
        out_shape=jax.ShapeDtypeStruct((N, D), jnp.float32),
    )(accs, dinv, loopc, w_in, w_out)


def kernel(x, edge_index, edge_type, rel_embed, w_in, w_out, w_loop, w_rel,
           loop_rel):
    N, D = x.shape
    Dh = D // 2
    R = rel_embed.shape[0]
    E = edge_index.shape[1]
    ne = E // 2

    Npad = _ceil_to(N + 1, 2048)
    Ep = _ceil_to(ne, _NS * 2 * _B)
    npad_e = Ep - ne
    Rpad = _ceil_to(R + 1, 8)

    pad_i = jnp.arange(npad_e, dtype=jnp.int32)
    pad_row = N + pad_i % (Npad - N)
    pad_col = pad_i % N
    pad_t = pad_i % (R + 1)

    rows, cols, ts = [], [], []
    for d in range(2):
        sl = slice(d * ne, (d + 1) * ne)
        rows.append(jnp.concatenate([edge_index[0, sl], pad_row]))
        cols.append(jnp.concatenate([edge_index[1, sl], pad_col]))
        ts.append(jnp.concatenate([edge_type[sl], pad_t]))
    grow = jnp.concatenate(rows).astype(jnp.int32)
    gt = jnp.concatenate(ts).astype(jnp.int32)
    gcol = jnp.concatenate(
        [cols[d] + d * Npad for d in range(2)]).astype(jnp.int32)

    bn = 2048
    Np2 = _ceil_to(N, bn)

    degc = _sc_degree(grow, Ep, Npad)
    cs, dinv, loopc, relout = _tc_prep(
        degc, x, rel_embed, loop_rel, w_rel, w_loop, Npad, Np2)
    xs = _tc_xs(x, dinv, Npad, Np2, bn).reshape(2 * Npad, D)
    accs = _sc_edges(gcol, grow, gt, xs, cs, Ep, Npad, Dh, Rpad)
    out = _tc_final(accs.reshape(4, Npad, D), dinv, loopc, w_in, w_out,
                    Npad, Np2, bn)
    return out, relout


# speedup vs baseline: 8.4426x; 1.6006x over previous
import functools
from math import gcd as _gcd

import jax
import jax.numpy as jnp
from jax import lax
from jax.experimental import pallas as pl
from jax.experimental.pallas import tpu as pltpu
from jax.experimental.pallas import tpu_sc as plsc

_PI = 3.141592653589793
_NC = 2
_NS = 16
_B = 64
_BE = 48
_ZR = 8


def _ceil_to(v, m):
    return (v + m - 1) // m * m


def _sc_degree(grow, Ep, Npad):
    nsl = Npad // _NS
    eps = Ep // _NS
    nblk = eps // _B
    mesh = plsc.VectorSubcoreMesh(core_axis_name="c", subcore_axis_name="s")

    @functools.partial(
        pl.kernel,
        out_type=jax.ShapeDtypeStruct((2 * Npad, 128), jnp.float32),
        mesh=mesh,
        scratch_types=[
            pltpu.VMEM_SHARED((Npad, 128), jnp.float32),
            pltpu.VMEM((_B,), jnp.int32),
            pltpu.VMEM((_B, 128), jnp.float32),
            pltpu.VMEM((_ZR, 128), jnp.float32),
        ],
    )
    def k(grow_hbm, out_hbm, acc, rowv, oneh, zb):
        d = lax.axis_index("c")
        s = lax.axis_index("s")
        lane = lax.iota(jnp.int32, 16)
        one16 = jnp.where(lane == 0, 1.0, 0.0).astype(jnp.float32)
        zero16 = jnp.zeros((16,), jnp.float32)

        @pl.loop(0, _B)
        def _(r):
            oneh[r, pl.ds(0, 16)] = one16
            for k8 in range(1, 8):
                oneh[r, pl.ds(k8 * 16, 16)] = zero16

        @pl.loop(0, _ZR)
        def _(r):
            for k8 in range(8):
                zb[r, pl.ds(k8 * 16, 16)] = zero16

        @pl.loop(0, nsl // _ZR)
        def _(j):
            pltpu.sync_copy(zb, acc.at[pl.ds(s * nsl + j * _ZR, _ZR)])

        plsc.subcore_barrier()

        base = d * Ep + s * eps

        @pl.loop(0, nblk)
        def _(b):
            pltpu.sync_copy(grow_hbm.at[pl.ds(base + b * _B, _B)], rowv)
            pltpu.sync_copy(oneh, acc.at[rowv], add=True)

        plsc.subcore_barrier()
        pltpu.sync_copy(
            acc.at[pl.ds(s * nsl, nsl)],
            out_hbm.at[pl.ds(d * Npad + s * nsl, nsl)],
        )

    return k(grow)


def _tc_prep(degc, x, rel_embed, loop_rel, w_rel, w_loop, Npad, Np2):
    N, D = x.shape
    Dh = D // 2
    R = rel_embed.shape[0]

    def body(degc_ref, x_ref, rel_ref, lrel_ref, wrel_ref, wloop_ref,
             cs_ref, dinv_ref, loopc_ref, relout_ref):
        deg = degc_ref[...][:, 0].reshape(2, Npad)[:, :N]
        dinv = jnp.where(deg > 0, lax.rsqrt(deg), 0.0)
        dinv_ref[...] = jnp.concatenate(
            [dinv, jnp.zeros((2, Np2 - N), jnp.float32)], axis=1)

        relf = jnp.concatenate([rel_ref[...], lrel_ref[...]], axis=0)
        r = relf * (_PI / 1.5)
        cs = jnp.concatenate([jnp.cos(r), jnp.sin(r)], axis=1)
        cs_ref[...] = jnp.concatenate(
            [cs, jnp.zeros((cs_ref.shape[0] - (R + 1), D), jnp.float32)], axis=0)

        xv = x_ref[...]
        cl = jnp.concatenate([cs[R, :Dh], cs[R, :Dh]], axis=0)
        sl = jnp.concatenate([cs[R, Dh:], cs[R, Dh:]], axis=0)
        wl = wloop_ref[...]
        wc = jnp.concatenate([wl[:Dh], -wl[Dh:]], axis=0)
        ws = jnp.concatenate([wl[Dh:], wl[:Dh]], axis=0)
        loopc_ref[...] = (
            jnp.dot(xv * cl[None, :], wc, preferred_element_type=jnp.float32)
            + jnp.dot(xv * sl[None, :], ws, preferred_element_type=jnp.float32))

        relout_ref[...] = jnp.dot(
            relf, wrel_ref[...], preferred_element_type=jnp.float32)[:R]

    Rpad = _ceil_to(R + 1, 8)
    return pl.pallas_call(
        body,
        out_shape=[
            jax.ShapeDtypeStruct((Rpad, D), jnp.float32),
            jax.ShapeDtypeStruct((2, Np2), jnp.float32),
            jax.ShapeDtypeStruct((N, D), jnp.float32),
            jax.ShapeDtypeStruct((R, Dh), jnp.float32),
        ],
    )(degc, x, rel_embed, loop_rel, w_rel, w_loop)


def _tc_xs(x, dinv, Npad, Np2, bn):
    N, D = x.shape

    def body(x_ref, dinv_ref, xs_ref):
        xv = x_ref[...]
        dv = dinv_ref[...]
        for d in range(2):
            xs_ref[d] = xv * dv[d][:, None]

    return pl.pallas_call(
        body,
        grid=(Np2 // bn,),
        in_specs=[
            pl.BlockSpec((bn, D), lambda i: (i, 0)),
            pl.BlockSpec((2, bn), lambda i: (0, i)),
        ],
        out_specs=pl.BlockSpec((2, bn, D), lambda i: (0, i, 0)),
        out_shape=jax.ShapeDtypeStruct((2, Npad, D), jnp.float32),
    )(x, dinv)


def _sc_edges(idx3, xs, cs, Ep, Npad, Dh, Rpad):
    nsl = Npad // _NS
    eps = Ep // _NS
    nblk = eps // _BE
    nblk2 = nblk // 2
    D = 2 * Dh
    mesh = plsc.VectorSubcoreMesh(core_axis_name="c", subcore_axis_name="s")

    @functools.partial(
        pl.kernel,
        out_type=jax.ShapeDtypeStruct((4 * Npad, D), jnp.float32),
        mesh=mesh,
        scratch_types=[
            pltpu.VMEM_SHARED((Npad, D), jnp.float32),
            pltpu.VMEM_SHARED((Rpad, D), jnp.float32),
            pltpu.VMEM((3, _BE), jnp.int32),
            pltpu.VMEM((3, _BE), jnp.int32),
            pltpu.VMEM((_BE, D), jnp.float32),
            pltpu.VMEM((_BE, D), jnp.float32),
            pltpu.VMEM((_BE, D), jnp.float32),
            pltpu.VMEM((_BE, D), jnp.float32),
            pltpu.VMEM((_BE, D), jnp.float32),
            pltpu.VMEM((_ZR, D), jnp.float32),
            pltpu.SemaphoreType.DMA,
            pltpu.SemaphoreType.DMA,
            pltpu.SemaphoreType.DMA,
            pltpu.SemaphoreType.DMA,
        ],
    )
    def k(idx_hbm, xs_hbm, cs_hbm, out_hbm, acc, css,
          idx0, idx1, xsv0, xsv1, csg0, csg1, pcs, zb,
          sg0, sg1, sc0, sc1):
        h = lax.axis_index("c")
        s = lax.axis_index("s")
        slots = ((idx0, xsv0, csg0, sg0, sc0),
                 (idx1, xsv1, csg1, sg1, sc1))

        @pl.when(s == 0)
        def _():
            pltpu.sync_copy(cs_hbm, css)

        zero16 = jnp.zeros((16,), jnp.float32)

        @pl.loop(0, _ZR)
        def _(r):
            for k8 in range(D // 16):
                zb[r, pl.ds(k8 * 16, 16)] = zero16

        def zero_own():
            @pl.loop(0, nsl // _ZR)
            def _(j):
                pltpu.sync_copy(zb, acc.at[pl.ds(s * nsl + j * _ZR, _ZR)])

        zero_own()
        plsc.subcore_barrier()

        def load_and_fire(fb, sl):
            idxg, xsv, csg, sg, sc = slots[sl]
            pltpu.sync_copy(idx_hbm.at[fb], idxg)
            pltpu.async_copy(xs_hbm.at[idxg.at[0]], xsv, sg)
            pltpu.async_copy(css.at[idxg.at[1]], csg, sc)

        def wait_gather(sl):
            _, xsv, csg, sg, sc = slots[sl]
            dummy = xs_hbm.at[pl.ds(0, _BE)]
            pltpu.make_async_copy(dummy, xsv, sg).wait()
            pltpu.make_async_copy(dummy, csg, sc).wait()

        def compute(sl, hoff):
            _, xsv, csg, _, _ = slots[sl]

            @pl.loop(0, _BE)
            def _(e):
                for kk in range(0, Dh, 16):
                    xv = xsv[e, pl.ds(hoff + kk, 16)]
                    cv = csg[e, pl.ds(kk, 16)]
                    sv = csg[e, pl.ds(Dh + kk, 16)]
                    pcs[e, pl.ds(kk, 16)] = xv * cv
                    pcs[e, pl.ds(Dh + kk, 16)] = xv * sv

        for d in range(2):
            base = (d * _NS + s) * nblk

            load_and_fire(base, 0)

            @pl.loop(0, nblk2)
            def _(p):
                for sl in range(2):
                    idxg, _, _, _, _ = slots[sl]
                    b = 2 * p + sl
                    wait_gather(sl)

                    @pl.when(b + 1 < nblk)
                    def _():
                        load_and_fire(base + b + 1, 1 - sl)

                    @pl.when(h == 0)
                    def _():
                        compute(sl, 0)

                    @pl.when(h == 1)
                    def _():
                        compute(sl, Dh)

                    pltpu.sync_copy(pcs, acc.at[idxg.at[2]], add=True)

            plsc.subcore_barrier()
            rowoff = (d * 2 + h) * Npad + s * nsl
            pltpu.sync_copy(acc.at[pl.ds(s * nsl, nsl)],
                            out_hbm.at[pl.ds(rowoff, nsl)])

            if d == 0:
                zero_own()
                plsc.subcore_barrier()

    return k(idx3, xs, cs)


def _tc_final(accs, dinv, loopc, w_in, w_out, Npad, Np2, bn):
    N, D = loopc.shape
    Dh = D // 2

    def body(acc_ref, dinv_ref, loopc_ref, win_ref, wout_ref, out_ref):
        a = acc_ref[...]
        dinv = dinv_ref[...]
        res = loopc_ref[...]
        for d, wref in ((0, win_ref), (1, wout_ref)):
            ac = jnp.concatenate([a[d * 2 + 0, :, :Dh], a[d * 2 + 1, :, :Dh]],
                                 axis=1)
            as_ = jnp.concatenate([a[d * 2 + 0, :, Dh:], a[d * 2 + 1, :, Dh:]],
                                  axis=1)
            w = wref[...]
            wc = jnp.concatenate([w[:Dh], -w[Dh:]], axis=0)
            ws = jnp.concatenate([w[Dh:], w[:Dh]], axis=0)
            contrib = (jnp.dot(ac, wc, preferred_element_type=jnp.float32)
                       + jnp.dot(as_, ws, preferred_element_type=jnp.float32))
            res = res + dinv[d][:, None] * contrib
        out_ref[...] = res * (1.0 / 3.0)

    return pl.pallas_call(
        body,
        grid=(Np2 // bn,),
        in_specs=[
            pl.BlockSpec((4, bn, D), lambda i: (0, i, 0)),
            pl.BlockSpec((2, bn), lambda i: (0, i)),
            pl.BlockSpec((bn, D), lambda i: (i, 0)),
            pl.BlockSpec((D, D), lambda i: (0, 0)),
            pl.BlockSpec((D, D), lambda i: (0, 0)),
        ],
        out_specs=pl.BlockSpec((bn, D), lambda i: (i, 0)),
        out_shape=jax.ShapeDtypeStruct((N, D), jnp.float32),
    )(accs, dinv, loopc, w_in, w_out)


def kernel(x, edge_index, edge_type, rel_embed, w_in, w_out, w_loop, w_rel,
           loop_rel):
    N, D = x.shape
    Dh = D // 2
    R = rel_embed.shape[0]
    E = edge_index.shape[1]
    ne = E // 2

    bn = 2048
    Npad = _ceil_to(N + 1, bn)
    Ep = _ceil_to(ne, _NS * 2 * _BE * _B // _gcd(2 * _BE, _B))
    npad_e = Ep - ne
    Rpad = _ceil_to(R + 1, 8)
    Np2 = Npad

    pad_i = jnp.arange(npad_e, dtype=jnp.int32)
    pad_row = N + pad_i % (Npad - N)
    pad_col = pad_i % N
    pad_t = pad_i % (R + 1)

    rows, cols, ts = [], [], []
    for d in range(2):
        sl = slice(d * ne, (d + 1) * ne)
        rows.append(jnp.concatenate([edge_index[0, sl], pad_row]))
        cols.append(jnp.concatenate([edge_index[1, sl], pad_col]))
        ts.append(jnp.concatenate([edge_type[sl], pad_t]))
    grow = jnp.concatenate(rows).astype(jnp.int32)

    nblk = Ep // _NS // _BE
    idx3 = jnp.stack(
        [jnp.stack([cols[d] + d * Npad, ts[d], rows[d]], axis=0)
         for d in range(2)]).astype(jnp.int32)
    idx3 = idx3.reshape(2, 3, _NS, nblk, _BE).transpose(0, 2, 3, 1, 4)
    idx3 = idx3.reshape(2 * _NS * nblk, 3, _BE)

    degc = _sc_degree(grow, Ep, Npad)
    cs, dinv, loopc, relout = _tc_prep(
        degc, x, rel_embed, loop_rel, w_rel, w_loop, Npad, Np2)
    xs = _tc_xs(x, dinv, Npad, Np2, bn).reshape(2 * Npad, D)
    accs = _sc_edges(idx3, xs, cs, Ep, Npad, Dh, Rpad)
    out = _tc_final(accs.reshape(4, Npad, D), dinv, loopc, w_in, w_out,
                    Npad, Np2, bn)
    return out, relout


# speedup vs baseline: 9.7105x; 1.1502x over previous
import functools
from math import gcd as _gcd

import jax
import jax.numpy as jnp
from jax import lax
from jax.experimental import pallas as pl
from jax.experimental.pallas import tpu as pltpu
from jax.experimental.pallas import tpu_sc as plsc

_PI = 3.141592653589793
_NC = 2
_NS = 16
_B = 64
_BE = 64
_ZR = 8


def _ceil_to(v, m):
    return (v + m - 1) // m * m


def _sc_degree(grow, Ep, Npad):
    nsl = Npad // _NS
    eps = Ep // _NS
    nblk = eps // _B
    mesh = plsc.VectorSubcoreMesh(core_axis_name="c", subcore_axis_name="s")

    @functools.partial(
        pl.kernel,
        out_type=jax.ShapeDtypeStruct((2 * Npad, 128), jnp.float32),
        mesh=mesh,
        scratch_types=[
            pltpu.VMEM_SHARED((Npad, 128), jnp.float32),
            pltpu.VMEM((_B,), jnp.int32),
            pltpu.VMEM((_B, 128), jnp.float32),
            pltpu.VMEM((_ZR, 128), jnp.float32),
        ],
    )
    def k(grow_hbm, out_hbm, acc, rowv, oneh, zb):
        d = lax.axis_index("c")
        s = lax.axis_index("s")
        lane = lax.iota(jnp.int32, 16)
        one16 = jnp.where(lane == 0, 1.0, 0.0).astype(jnp.float32)
        zero16 = jnp.zeros((16,), jnp.float32)

        @pl.loop(0, _B)
        def _(r):
            oneh[r, pl.ds(0, 16)] = one16
            for k8 in range(1, 8):
                oneh[r, pl.ds(k8 * 16, 16)] = zero16

        @pl.loop(0, _ZR)
        def _(r):
            for k8 in range(8):
                zb[r, pl.ds(k8 * 16, 16)] = zero16

        @pl.loop(0, nsl // _ZR)
        def _(j):
            pltpu.sync_copy(zb, acc.at[pl.ds(s * nsl + j * _ZR, _ZR)])

        plsc.subcore_barrier()

        base = d * Ep + s * eps

        @pl.loop(0, nblk)
        def _(b):
            pltpu.sync_copy(grow_hbm.at[pl.ds(base + b * _B, _B)], rowv)
            pltpu.sync_copy(oneh, acc.at[rowv], add=True)

        plsc.subcore_barrier()
        pltpu.sync_copy(
            acc.at[pl.ds(s * nsl, nsl)],
            out_hbm.at[pl.ds(d * Npad + s * nsl, nsl)],
        )

    return k(grow)


def _tc_prep(degc, x, rel_embed, loop_rel, w_rel, w_loop, Npad, Np2):
    N, D = x.shape
    Dh = D // 2
    R = rel_embed.shape[0]

    def body(degc_ref, x_ref, rel_ref, lrel_ref, wrel_ref, wloop_ref,
             cs_ref, dinv_ref, loopc_ref, relout_ref):
        deg = degc_ref[...][:, 0].reshape(2, Npad)[:, :N]
        dinv = jnp.where(deg > 0, lax.rsqrt(deg), 0.0)
        dinv_ref[...] = jnp.concatenate(
            [dinv, jnp.zeros((2, Np2 - N), jnp.float32)], axis=1)

        relf = jnp.concatenate([rel_ref[...], lrel_ref[...]], axis=0)
        r = relf * (_PI / 1.5)
        cs = jnp.concatenate([jnp.cos(r), jnp.sin(r)], axis=1)
        cs_ref[...] = jnp.concatenate(
            [cs, jnp.zeros((cs_ref.shape[0] - (R + 1), D), jnp.float32)], axis=0)

        xv = x_ref[...]
        cl = jnp.concatenate([cs[R, :Dh], cs[R, :Dh]], axis=0)
        sl = jnp.concatenate([cs[R, Dh:], cs[R, Dh:]], axis=0)
        wl = wloop_ref[...]
        wc = jnp.concatenate([wl[:Dh], -wl[Dh:]], axis=0)
        ws = jnp.concatenate([wl[Dh:], wl[:Dh]], axis=0)
        loopc_ref[...] = (
            jnp.dot(xv * cl[None, :], wc, preferred_element_type=jnp.float32)
            + jnp.dot(xv * sl[None, :], ws, preferred_element_type=jnp.float32))

        relout_ref[...] = jnp.dot(
            relf, wrel_ref[...], preferred_element_type=jnp.float32)[:R]

    Rpad = _ceil_to(R + 1, 8)
    return pl.pallas_call(
        body,
        out_shape=[
            jax.ShapeDtypeStruct((Rpad, D), jnp.float32),
            jax.ShapeDtypeStruct((2, Np2), jnp.float32),
            jax.ShapeDtypeStruct((N, D), jnp.float32),
            jax.ShapeDtypeStruct((R, Dh), jnp.float32),
        ],
    )(degc, x, rel_embed, loop_rel, w_rel, w_loop)


def _tc_xs(x, dinv, Npad, Np2, bn):
    N, D = x.shape

    def body(x_ref, dinv_ref, xs_ref):
        xv = x_ref[...]
        dv = dinv_ref[...]
        for d in range(2):
            xs_ref[d] = xv * dv[d][:, None]

    return pl.pallas_call(
        body,
        grid=(Np2 // bn,),
        in_specs=[
            pl.BlockSpec((bn, D), lambda i: (i, 0)),
            pl.BlockSpec((2, bn), lambda i: (0, i)),
        ],
        out_specs=pl.BlockSpec((2, bn, D), lambda i: (0, i, 0)),
        out_shape=jax.ShapeDtypeStruct((2, Npad, D), jnp.float32),
    )(x, dinv)


def _sc_edges(idx3, xs, cs, Ep, Npad, Dh, Rpad):
    nsl = Npad // _NS
    eps = Ep // _NS
    nblk = eps // _BE
    nblk2 = nblk // 2
    D = 2 * Dh
    mesh = plsc.VectorSubcoreMesh(core_axis_name="c", subcore_axis_name="s")

    @functools.partial(
        pl.kernel,
        out_type=jax.ShapeDtypeStruct((4 * Npad, D), jnp.float32),
        mesh=mesh,
        scratch_types=[
            pltpu.VMEM_SHARED((Npad, D), jnp.float32),
            pltpu.VMEM_SHARED((Rpad, D), jnp.float32),
            pltpu.VMEM((3, _BE), jnp.int32),
            pltpu.VMEM((3, _BE), jnp.int32),
            pltpu.VMEM((_BE, D), jnp.float32),
            pltpu.VMEM((_BE, D), jnp.float32),
            pltpu.VMEM((_BE, D), jnp.float32),
            pltpu.VMEM((_BE, D), jnp.float32),
            pltpu.VMEM((_BE, D), jnp.float32),
            pltpu.VMEM((_ZR, D), jnp.float32),
            pltpu.SemaphoreType.DMA,
            pltpu.SemaphoreType.DMA,
            pltpu.SemaphoreType.DMA,
            pltpu.SemaphoreType.DMA,
        ],
    )
    def k(idx_hbm, xs_hbm, cs_hbm, out_hbm, acc, css,
          idx0, idx1, xsv0, xsv1, csg0, csg1, pcs, zb,
          sg0, sg1, sc0, sc1):
        h = lax.axis_index("c")
        s = lax.axis_index("s")
        slots = ((idx0, xsv0, csg0, sg0, sc0),
                 (idx1, xsv1, csg1, sg1, sc1))

        @pl.when(s == 0)
        def _():
            pltpu.sync_copy(cs_hbm, css)

        zero16 = jnp.zeros((16,), jnp.float32)

        @pl.loop(0, _ZR)
        def _(r):
            for k8 in range(D // 16):
                zb[r, pl.ds(k8 * 16, 16)] = zero16

        def zero_own():
            @pl.loop(0, nsl // _ZR)
            def _(j):
                pltpu.sync_copy(zb, acc.at[pl.ds(s * nsl + j * _ZR, _ZR)])

        zero_own()
        plsc.subcore_barrier()

        def load_and_fire(fb, sl):
            idxg, xsv, csg, sg, sc = slots[sl]
            pltpu.sync_copy(idx_hbm.at[fb], idxg)
            pltpu.async_copy(xs_hbm.at[idxg.at[0]], xsv, sg)
            pltpu.async_copy(css.at[idxg.at[1]], csg, sc)

        def wait_gather(sl):
            _, xsv, csg, sg, sc = slots[sl]
            dummy = xs_hbm.at[pl.ds(0, _BE)]
            pltpu.make_async_copy(dummy, xsv, sg).wait()
            pltpu.make_async_copy(dummy, csg, sc).wait()

        def compute(sl, hoff):
            _, xsv, csg, _, _ = slots[sl]

            @pl.loop(0, _BE)
            def _(e):
                for kk in range(0, Dh, 16):
                    xv = xsv[e, pl.ds(hoff + kk, 16)]
                    cv = csg[e, pl.ds(kk, 16)]
                    sv = csg[e, pl.ds(Dh + kk, 16)]
                    pcs[e, pl.ds(kk, 16)] = xv * cv
                    pcs[e, pl.ds(Dh + kk, 16)] = xv * sv

        for d in range(2):
            base = (d * _NS + s) * nblk

            load_and_fire(base, 0)

            @pl.loop(0, nblk2)
            def _(p):
                for sl in range(2):
                    idxg, _, _, _, _ = slots[sl]
                    b = 2 * p + sl
                    wait_gather(sl)

                    @pl.when(b + 1 < nblk)
                    def _():
                        load_and_fire(base + b + 1, 1 - sl)

                    @pl.when(h == 0)
                    def _():
                        compute(sl, 0)

                    @pl.when(h == 1)
                    def _():
                        compute(sl, Dh)

                    pltpu.sync_copy(pcs, acc.at[idxg.at[2]], add=True)

            plsc.subcore_barrier()
            rowoff = (d * 2 + h) * Npad + s * nsl
            pltpu.sync_copy(acc.at[pl.ds(s * nsl, nsl)],
                            out_hbm.at[pl.ds(rowoff, nsl)])

            if d == 0:
                zero_own()
                plsc.subcore_barrier()

    return k(idx3, xs, cs)


def _tc_final(accs, dinv, loopc, w_in, w_out, Npad, Np2, bn):
    N, D = loopc.shape
    Dh = D // 2

    def body(acc_ref, dinv_ref, loopc_ref, win_ref, wout_ref, out_ref):
        a = acc_ref[...]
        dinv = dinv_ref[...]
        res = loopc_ref[...]
        for d, wref in ((0, win_ref), (1, wout_ref)):
            ac = jnp.concatenate([a[d * 2 + 0, :, :Dh], a[d * 2 + 1, :, :Dh]],
                                 axis=1)
            as_ = jnp.concatenate([a[d * 2 + 0, :, Dh:], a[d * 2 + 1, :, Dh:]],
                                  axis=1)
            w = wref[...]
            wc = jnp.concatenate([w[:Dh], -w[Dh:]], axis=0)
            ws = jnp.concatenate([w[Dh:], w[:Dh]], axis=0)
            contrib = (jnp.dot(ac, wc, preferred_element_type=jnp.float32)
                       + jnp.dot(as_, ws, preferred_element_type=jnp.float32))
            res = res + dinv[d][:, None] * contrib
        out_ref[...] = res * (1.0 / 3.0)

    return pl.pallas_call(
        body,
        grid=(Np2 // bn,),
        in_specs=[
            pl.BlockSpec((4, bn, D), lambda i: (0, i, 0)),
            pl.BlockSpec((2, bn), lambda i: (0, i)),
            pl.BlockSpec((bn, D), lambda i: (i, 0)),
            pl.BlockSpec((D, D), lambda i: (0, 0)),
            pl.BlockSpec((D, D), lambda i: (0, 0)),
        ],
        out_specs=pl.BlockSpec((bn, D), lambda i: (i, 0)),
        out_shape=jax.ShapeDtypeStruct((N, D), jnp.float32),
    )(accs, dinv, loopc, w_in, w_out)


def kernel(x, edge_index, edge_type, rel_embed, w_in, w_out, w_loop, w_rel,
           loop_rel):
    N, D = x.shape
    Dh = D // 2
    R = rel_embed.shape[0]
    E = edge_index.shape[1]
    ne = E // 2

    bn = 2048
    Npad = _ceil_to(N + 1, bn)
    Ep = _ceil_to(ne, _NS * 2 * _BE * _B // _gcd(2 * _BE, _B))
    npad_e = Ep - ne
    Rpad = _ceil_to(R + 1, 8)
    Np2 = Npad

    pad_i = jnp.arange(npad_e, dtype=jnp.int32)
    pad_row = N + pad_i % (Npad - N)
    pad_col = pad_i % N
    pad_t = pad_i % (R + 1)

    rows, cols, ts = [], [], []
    for d in range(2):
        sl = slice(d * ne, (d + 1) * ne)
        rows.append(jnp.concatenate([edge_index[0, sl], pad_row]))
        cols.append(jnp.concatenate([edge_index[1, sl], pad_col]))
        ts.append(jnp.concatenate([edge_type[sl], pad_t]))
    grow = jnp.concatenate(rows).astype(jnp.int32)

    nblk = Ep // _NS // _BE
    idx3 = jnp.stack(
        [jnp.stack([cols[d] + d * Npad, ts[d], rows[d]], axis=0)
         for d in range(2)]).astype(jnp.int32)
    idx3 = idx3.reshape(2, 3, _NS, nblk, _BE).transpose(0, 2, 3, 1, 4)
    idx3 = idx3.reshape(2 * _NS * nblk, 3, _BE)

    degc = _sc_degree(grow, Ep, Npad)
    cs, dinv, loopc, relout = _tc_prep(
        degc, x, rel_embed, loop_rel, w_rel, w_loop, Npad, Np2)
    xs = _tc_xs(x, dinv, Npad, Np2, bn).reshape(2 * Npad, D)
    accs = _sc_edges(idx3, xs, cs, Ep, Npad, Dh, Rpad)
    out = _tc_final(accs.reshape(4, Npad, D), dinv, loopc, w_in, w_out,
                    Npad, Np2, bn)
    return out, relout
